# baseline (speedup 1.0000x reference)
import functools

import jax
import jax.numpy as jnp
from jax import lax
from jax.experimental import pallas as pl
from jax.experimental.pallas import tpu as pltpu
from jax.experimental.pallas import tpu_sc as plsc

VOCAB = 1000
D = 1000
B_TOTAL = 4 * 2048
NC, NS = 2, 16
NW = NC * NS
B_PER_W = B_TOTAL // NW
CHUNK = 32
NBUF = 4
NCHUNKS = B_PER_W // CHUNK

_MESH = plsc.VectorSubcoreMesh(
    core_axis_name="c", subcore_axis_name="s", num_cores=NC, num_subcores=NS)


@functools.partial(
    pl.kernel,
    out_type=jax.ShapeDtypeStruct((B_TOTAL, D), jnp.float32),
    mesh=_MESH,
    compiler_params=pltpu.CompilerParams(use_tc_tiling_on_sc=False),
    scratch_types=[
        pltpu.VMEM((NCHUNKS, CHUNK), jnp.int32),
        pltpu.VMEM((NBUF, CHUNK, D), jnp.float32),
        pltpu.SemaphoreType.DMA,
        pltpu.SemaphoreType.DMA,
    ],
)
def _gather_rows(idx_hbm, table_hbm, out_hbm, idx_v, rows_v, gsem, ssem):
    wid = lax.axis_index("s") * NC + lax.axis_index("c")
    base = wid * B_PER_W
    pltpu.sync_copy(idx_hbm.at[wid], idx_v)

    gathers = [None] * NCHUNKS
    scatters = [None] * NCHUNKS
    s_waited = [False] * NCHUNKS
    for ch in range(min(NBUF, NCHUNKS)):
        gathers[ch] = pltpu.async_copy(
            table_hbm.at[idx_v.at[ch]], rows_v.at[ch], gsem)
    for ch in range(NCHUNKS):
        gathers[ch].wait()
        scatters[ch] = pltpu.async_copy(
            rows_v.at[ch % NBUF],
            out_hbm.at[pl.ds(base + ch * CHUNK, CHUNK)], ssem)
        prev = ch - (NBUF - 1)
        nxt = ch + 1
        if prev >= 0 and nxt < NCHUNKS and gathers[nxt] is None:
            scatters[prev].wait()
            s_waited[prev] = True
            gathers[nxt] = pltpu.async_copy(
                table_hbm.at[idx_v.at[nxt]], rows_v.at[nxt % NBUF], gsem)
    for ch in range(NCHUNKS):
        if not s_waited[ch]:
            scatters[ch].wait()


def kernel(idx, targets, table):
    del targets
    idx3 = idx.reshape(NW, NCHUNKS, CHUNK).astype(jnp.int32)
    out = _gather_rows(idx3, table)
    return out.reshape(4, 2048, VOCAB)


# speedup vs baseline: 1.0002x; 1.0002x over previous
import functools

import jax
import jax.numpy as jnp
from jax import lax
from jax.experimental import pallas as pl
from jax.experimental.pallas import tpu as pltpu
from jax.experimental.pallas import tpu_sc as plsc

VOCAB = 1000
D = 1000
B, T = 4, 2048
NC, NS = 2, 16
NW = NC * NS
B_PER_W = (B * T) // NW
CHUNK = 32
NBUF = 4
NCHUNKS = B_PER_W // CHUNK
W_PER_B = T // B_PER_W

_MESH = plsc.VectorSubcoreMesh(
    core_axis_name="c", subcore_axis_name="s", num_cores=NC, num_subcores=NS)


@functools.partial(
    pl.kernel,
    out_type=jax.ShapeDtypeStruct((B, T, D), jnp.float32),
    mesh=_MESH,
    compiler_params=pltpu.CompilerParams(use_tc_tiling_on_sc=False),
    scratch_types=[
        pltpu.VMEM((B_PER_W,), jnp.int32),
        pltpu.VMEM((NBUF, CHUNK, D), jnp.float32),
        pltpu.SemaphoreType.DMA,
        pltpu.SemaphoreType.DMA,
    ],
)
def _gather_rows(idx_hbm, table_hbm, out_hbm, idx_v, rows_v, gsem, ssem):
    wid = lax.axis_index("s") * NC + lax.axis_index("c")
    b = wid // W_PER_B
    t0 = (wid % W_PER_B) * B_PER_W
    pltpu.sync_copy(idx_hbm.at[b, pl.ds(t0, B_PER_W)], idx_v)

    gathers = [None] * NCHUNKS
    scatters = [None] * NCHUNKS
    s_waited = [False] * NCHUNKS
    for ch in range(min(NBUF, NCHUNKS)):
        gathers[ch] = pltpu.async_copy(
            table_hbm.at[idx_v.at[pl.ds(ch * CHUNK, CHUNK)]],
            rows_v.at[ch], gsem)
    for ch in range(NCHUNKS):
        gathers[ch].wait()
        scatters[ch] = pltpu.async_copy(
            rows_v.at[ch % NBUF],
            out_hbm.at[b, pl.ds(t0 + ch * CHUNK, CHUNK)], ssem)
        prev = ch - (NBUF - 1)
        nxt = ch + 1
        if prev >= 0 and nxt < NCHUNKS and gathers[nxt] is None:
            scatters[prev].wait()
            s_waited[prev] = True
            gathers[nxt] = pltpu.async_copy(
                table_hbm.at[idx_v.at[pl.ds(nxt * CHUNK, CHUNK)]],
                rows_v.at[nxt % NBUF], gsem)
    for ch in range(NCHUNKS):
        if not s_waited[ch]:
            scatters[ch].wait()


def kernel(idx, targets, table):
    del targets
    if idx.dtype != jnp.int32:
        idx = idx.astype(jnp.int32)
    return _gather_rows(idx, table)


# speedup vs baseline: 1.4654x; 1.4652x over previous
import functools

import jax
import jax.numpy as jnp
from jax import lax
from jax.experimental import pallas as pl
from jax.experimental.pallas import tpu as pltpu
from jax.experimental.pallas import tpu_sc as plsc

VOCAB = 1000
D = 1000
DP = 1024
B, T = 4, 2048
NC, NS = 2, 16
NW = NC * NS
B_PER_W = (B * T) // NW
CHUNK = 32
NBUF = 3
NCHUNKS = B_PER_W // CHUNK
W_PER_B = T // B_PER_W

_MESH = plsc.VectorSubcoreMesh(
    core_axis_name="c", subcore_axis_name="s", num_cores=NC, num_subcores=NS)


@functools.partial(
    pl.kernel,
    out_type=jax.ShapeDtypeStruct((B, T, DP), jnp.float32),
    mesh=_MESH,
    scratch_types=[
        pltpu.VMEM((B_PER_W,), jnp.int32),
        pltpu.VMEM((NBUF, CHUNK, DP), jnp.float32),
        pltpu.SemaphoreType.DMA,
        pltpu.SemaphoreType.DMA,
    ],
)
def _gather_rows(idx_hbm, table_hbm, out_hbm, idx_v, rows_v, gsem, ssem):
    wid = lax.axis_index("s") * NC + lax.axis_index("c")
    b = wid // W_PER_B
    t0 = (wid % W_PER_B) * B_PER_W
    pltpu.sync_copy(idx_hbm.at[pl.ds(wid * B_PER_W, B_PER_W)], idx_v)

    gathers = [None] * NCHUNKS
    scatters = [None] * NCHUNKS
    s_waited = [False] * NCHUNKS
    for ch in range(min(NBUF, NCHUNKS)):
        gathers[ch] = pltpu.async_copy(
            table_hbm.at[idx_v.at[pl.ds(ch * CHUNK, CHUNK)]],
            rows_v.at[ch], gsem)
    for ch in range(NCHUNKS):
        gathers[ch].wait()
        scatters[ch] = pltpu.async_copy(
            rows_v.at[ch % NBUF],
            out_hbm.at[b, pl.ds(t0 + ch * CHUNK, CHUNK)], ssem)
        prev = ch - (NBUF - 1)
        nxt = ch + 1
        if prev >= 0 and nxt < NCHUNKS and gathers[nxt] is None:
            scatters[prev].wait()
            s_waited[prev] = True
            gathers[nxt] = pltpu.async_copy(
                table_hbm.at[idx_v.at[pl.ds(nxt * CHUNK, CHUNK)]],
                rows_v.at[nxt % NBUF], gsem)
    for ch in range(NCHUNKS):
        if not s_waited[ch]:
            scatters[ch].wait()


def kernel(idx, targets, table):
    del targets
    idx_flat = idx.reshape(B * T)
    if idx_flat.dtype != jnp.int32:
        idx_flat = idx_flat.astype(jnp.int32)
    table_pad = jnp.pad(table, ((0, 0), (0, DP - D)))
    out_pad = _gather_rows(idx_flat, table_pad)
    return out_pad[:, :, :D]
